# baseline (speedup 1.0000x reference)
import functools

import jax
import jax.numpy as jnp
from jax import lax
from jax.experimental import pallas as pl
from jax.experimental.pallas import tpu as pltpu
from jax.experimental.pallas import tpu_sc as plsc

N, E, D, ED, B, NT = 10000, 20000, 128, 4, 512, 8
EP = 20480
NP = 10240
TIP = 12288
DUMMY_ROW = 10200

_CP = pltpu.CompilerParams(vmem_limit_bytes=100 * 1024 * 1024)


_HI = lax.Precision.HIGHEST


def _r(x):
    return x.astype(jnp.bfloat16).astype(jnp.float32)


def _rb(x):
    return x.astype(jnp.bfloat16)


def _lin_bn_body(*refs, act, nrows, n_in, rc):
    a_refs = refs[:n_in]
    w_refs = refs[n_in:2 * n_in]
    g_ref, b_ref, o_ref, y_s = refs[2 * n_in:]
    ws = [_rb(w_ref[...]) for w_ref in w_refs]
    nch = nrows // rc

    def pass1(ci, carry):
        s, ss = carry
        y = jnp.dot(_rb(a_refs[0][pl.ds(ci * rc, rc), :]), ws[0],
                    preferred_element_type=jnp.float32)
        for j in range(1, n_in):
            y = y + jnp.dot(_rb(a_refs[j][pl.ds(ci * rc, rc), :]), ws[j],
                            preferred_element_type=jnp.float32)
        y_s[pl.ds(ci * rc, rc), :] = y
        return (s + jnp.sum(y, axis=0, keepdims=True),
                ss + jnp.sum(y * y, axis=0, keepdims=True))

    ct = ws[0].shape[1]
    s, ss = lax.fori_loop(0, nch, pass1, (jnp.zeros((1, ct), jnp.float32),
                                          jnp.zeros((1, ct), jnp.float32)))
    m = s / float(nrows)
    v = ss / float(nrows) - m * m
    sc = g_ref[...] / jnp.sqrt(v + 1e-5)
    off = b_ref[...] - m * sc

    def pass2(ci, _):
        y = y_s[pl.ds(ci * rc, rc), :] * sc + off
        if act:
            y = jnp.maximum(y, 0.0)
        o_ref[pl.ds(ci * rc, rc), :] = y
        return 0

    lax.fori_loop(0, nch, pass2, 0)


def _lin_bn_multi(a_list, w_list, g, b, act, col_tile=128, rc=2000):
    rows = a_list[0].shape[0]
    cols = w_list[0].shape[1]
    nt = cols // col_tile
    n_in = len(a_list)
    in_specs = ([pl.BlockSpec(a.shape, lambda i: (0, 0)) for a in a_list]
                + [pl.BlockSpec((w.shape[0], col_tile), lambda i: (0, i))
                   for w in w_list]
                + [pl.BlockSpec((1, col_tile), lambda i: (0, i)),
                   pl.BlockSpec((1, col_tile), lambda i: (0, i))])
    return pl.pallas_call(
        functools.partial(_lin_bn_body, act=act, nrows=rows, n_in=n_in, rc=rc),
        grid=(nt,),
        in_specs=in_specs,
        out_specs=pl.BlockSpec((rows, col_tile), lambda i: (0, i)),
        out_shape=jax.ShapeDtypeStruct((rows, cols), jnp.float32),
        scratch_shapes=[pltpu.VMEM((rows, col_tile), jnp.float32)],
        compiler_params=_CP,
    )(*a_list, *w_list, g.reshape(1, -1), b.reshape(1, -1))


def _lin_bn(a, w, g, b, act, col_tile=128, rc=2000):
    return _lin_bn_multi([a], [w], g, b, act, col_tile, rc)


def _stats_body(e3_ref, ebar_ref, gram_ref):
    rc = 2000
    nch = E // rc

    def acc(ci, carry):
        s, gm = carry
        e3 = _r(e3_ref[pl.ds(ci * rc, rc), :])
        return (s + jnp.sum(e3, axis=0, keepdims=True),
                gm + lax.dot_general(e3, e3, (((0,), (0,)), ((), ())),
                                     preferred_element_type=jnp.float32,
                                     precision=_HI))

    s, gm = lax.fori_loop(0, nch, acc, (jnp.zeros((1, D), jnp.float32),
                                        jnp.zeros((D, D), jnp.float32)))
    ebar_ref[...] = s / float(E)
    gram_ref[...] = gm / float(E)


def _enc4_body(ebar_ref, gram_ref, w4_ref, g4_ref, b4_ref,
               tp2_ref, s4_ref, cmat_ref):
    ebar = ebar_ref[...]
    cov = gram_ref[...] - lax.dot_general(
        ebar, ebar, (((0,), (0,)), ((), ())),
        preferred_element_type=jnp.float32, precision=_HI)
    w4 = _r(w4_ref[...])
    m4 = jnp.dot(ebar, w4, preferred_element_type=jnp.float32, precision=_HI)
    q = jnp.dot(cov, w4, preferred_element_type=jnp.float32, precision=_HI)
    v4 = jnp.sum(q * w4, axis=0, keepdims=True)
    s4 = g4_ref[...] / jnp.sqrt(v4 + 1e-5)
    tp2_ref[...] = w4
    s4_ref[...] = s4
    cmat_ref[...] = (b4_ref[...] - m4 * s4).reshape(32, D)


def _msg_body(e3_ref, ost_ref, tp2_ref, s4_ref, cmat_ref, msg_ref):
    te = ost_ref.shape[0]
    ost = _r(ost_ref[...])
    p = jnp.dot(_rb(e3_ref[...]), _rb(tp2_ref[...]),
                preferred_element_type=jnp.float32)
    s43 = s4_ref[...].reshape(1, D, D)
    cm3 = cmat_ref[...].reshape(1, D, D)
    w_e = _r(p.reshape(te, D, D) * s43 + cm3)
    msg_ref[...] = jnp.sum(w_e * ost[:, :, None], axis=1)


def _premsg_body(pp_ref, cp_ref, cb_ref, m_ref):
    cnt = jnp.maximum(cp_ref[...], 1.0)
    m_ref[...] = jnp.maximum(pp_ref[...] / cnt + cb_ref[...], 0.0)


def _gru_body(m_ref, h_ref, wih2_ref, whh2_ref, b2_ref, wihn_ref, whhn_ref,
              bihn_ref, bhhn_ref, o_ref):
    m = _rb(m_ref[...])
    h = h_ref[...]
    hb = _rb(h)
    g2 = (jnp.dot(m, _rb(wih2_ref[...]), preferred_element_type=jnp.float32)
          + jnp.dot(hb, _rb(whh2_ref[...]), preferred_element_type=jnp.float32)
          + b2_ref[...])
    r = jax.nn.sigmoid(g2[:, :D])
    z = jax.nn.sigmoid(g2[:, D:])
    gxn = jnp.dot(m, _rb(wihn_ref[...]), preferred_element_type=jnp.float32) + bihn_ref[...]
    ghn = jnp.dot(hb, _rb(whhn_ref[...]), preferred_element_type=jnp.float32) + bhhn_ref[...]
    n = jnp.tanh(gxn + r * ghn)
    o_ref[...] = (1.0 - z) * n + z * h


_S2S_NC = 2000


def _set2set_body(out_ref, batch_ref, wih_ref, whh_ref, bih_ref, bhh_ref,
                  qs_ref, ee_s, ex_s):
    f32 = jnp.float32
    nc = _S2S_NC
    nch = N // nc
    iota_b = lax.broadcasted_iota(jnp.int32, (nc, B), 1)

    def _mask(ci):
        bat = batch_ref[pl.ds(ci * nc, nc), :]
        return (bat == iota_b).astype(f32)

    hl = jnp.zeros((B, D), f32)
    cl = jnp.zeros((B, D), f32)
    q_star = jnp.zeros((B, 2 * D), f32)
    for _ in range(6):
        g_all = (jnp.dot(_rb(q_star), _rb(wih_ref[...]), preferred_element_type=f32)
                 + bih_ref[...]
                 + jnp.dot(_rb(hl), _rb(whh_ref[...]), preferred_element_type=f32)
                 + bhh_ref[...])
        ig = jax.nn.sigmoid(g_all[:, :D])
        fg = jax.nn.sigmoid(g_all[:, D:2 * D])
        gg = jnp.tanh(g_all[:, 2 * D:3 * D])
        og = jax.nn.sigmoid(g_all[:, 3 * D:])
        cl = fg * cl + ig * gg
        hl = og * jnp.tanh(cl)
        q = hl

        def p1(ci, emax):
            m = _mask(ci)
            outc = out_ref[pl.ds(ci * nc, nc), :]
            qb = jnp.dot(m, q, preferred_element_type=f32, precision=lax.Precision.HIGHEST)
            ee = jnp.sum(outc * qb, axis=1, keepdims=True)
            ee_s[pl.ds(ci * nc, nc), :] = ee
            vals = m * ee + (m - 1.0) * 1e30
            return jnp.maximum(emax, jnp.max(vals, axis=0, keepdims=True))

        emax = lax.fori_loop(0, nch, p1, jnp.full((1, B), -1e30, f32))

        def p2(ci, esum):
            m = _mask(ci)
            emaxb = jnp.sum(m * emax, axis=1, keepdims=True)
            ex = jnp.exp(ee_s[pl.ds(ci * nc, nc), :] - emaxb)
            ex_s[pl.ds(ci * nc, nc), :] = ex
            return esum + jnp.sum(m * ex, axis=0, keepdims=True)

        esum = lax.fori_loop(0, nch, p2, jnp.zeros((1, B), f32))

        def p3(ci, rr):
            m = _mask(ci)
            esb = jnp.sum(m * esum, axis=1, keepdims=True)
            a = ex_s[pl.ds(ci * nc, nc), :] / (esb + 1e-16)
            outc = out_ref[pl.ds(ci * nc, nc), :]
            return rr + lax.dot_general(m, a * outc, (((0,), (0,)), ((), ())),
                                        preferred_element_type=f32, precision=lax.Precision.HIGHEST)

        rr = lax.fori_loop(0, nch, p3, jnp.zeros((B, D), f32))
        q_star = jnp.concatenate([q, rr], axis=1)
    qs_ref[...] = q_star


def _pred3_body(a_ref, w_ref, b_ref, tc_ref, o_ref):
    rows = a_ref.shape[0]
    y = jnp.dot(_rb(a_ref[...]), _rb(w_ref[...]),
                preferred_element_type=jnp.float32) + b_ref[...]
    sel = (tc_ref[...] == lax.broadcasted_iota(jnp.int32, (rows, D), 1))
    o_ref[...] = jnp.sum(jnp.where(sel, y, 0.0), axis=1, keepdims=True)


_MESH = plsc.VectorSubcoreMesh(core_axis_name="c", subcore_axis_name="s")


def _sc_gather(table, idx3, nj, dm):
    rows_out = 32 * nj * 128

    @functools.partial(
        pl.kernel, mesh=_MESH,
        out_type=jax.ShapeDtypeStruct((rows_out, dm), jnp.float32),
        scratch_types=[
            pltpu.VMEM((nj, 128), jnp.int32),
            pltpu.VMEM((nj * 128, dm), jnp.float32),
            pltpu.SemaphoreType.DMA,
        ],
    )
    def gk(table_hbm, idx_hbm, out_hbm, idx_v, rows_v, sem):
        c = lax.axis_index("c")
        s = lax.axis_index("s")
        wid = s * 2 + c
        pltpu.sync_copy(idx_hbm.at[wid], idx_v)
        cps = [pltpu.async_copy(table_hbm.at[idx_v.at[j]],
                                rows_v.at[pl.ds(j * 128, 128)], sem)
               for j in range(nj)]
        for cp in cps:
            cp.wait()
        pltpu.sync_copy(rows_v, out_hbm.at[pl.ds(wid * nj * 128, nj * 128)])

    return gk(table, idx3)


NPH = 5120
SPAD = 5248


def _sc_scatter_add(vals, dst3, zeros328):
    @functools.partial(
        pl.kernel, mesh=_MESH,
        out_type=jax.ShapeDtypeStruct((2, NPH, 128), jnp.float32),
        scratch_types=[
            pltpu.VMEM((5, 128), jnp.int32),
            pltpu.VMEM((640, 128), jnp.float32),
            pltpu.VMEM_SHARED((SPAD, 128), jnp.float32),
        ],
    )
    def sk(vals_hbm, dst_hbm, z_hbm, out_hbm, idx_v, buf, shared):
        c = lax.axis_index("c")
        s = lax.axis_index("s")
        lo = c * NPH
        pltpu.sync_copy(z_hbm, shared.at[pl.ds(s * 328, 328)])
        plsc.subcore_barrier()
        for half in range(2):
            r = half * 16 + s
            pltpu.sync_copy(dst_hbm.at[r], idx_v)
            for j in range(5):
                for l in range(8):
                    v = idx_v[j, pl.ds(l * 16, 16)] - lo
                    ok = (v >= 0) & (v < NPH)
                    idx_v[j, pl.ds(l * 16, 16)] = jnp.where(ok, v, NPH)
            pltpu.sync_copy(vals_hbm.at[pl.ds(r * 640, 640)], buf)
            for j in range(5):
                pltpu.sync_copy(buf.at[pl.ds(j * 128, 128)],
                                shared.at[idx_v.at[j]], add=True)
        plsc.subcore_barrier()
        pltpu.sync_copy(shared.at[pl.ds(s * 320, 320)], buf.at[pl.ds(0, 320)])
        pltpu.sync_copy(buf.at[pl.ds(0, 320)], out_hbm.at[c, pl.ds(s * 320, 320)])

    return sk(vals, dst3, zeros328)


def _full_call(body, out_shapes, *args):
    return pl.pallas_call(body, out_shape=out_shapes, compiler_params=_CP)(*args)


def kernel(x, edge_attr, params, edge_index, target_index, batch, target_class):
    p = params
    f32, i32 = jnp.float32, jnp.int32

    src_p = jnp.concatenate([edge_index[0].astype(i32),
                             jnp.zeros((EP - E,), i32)]).reshape(32, 5, 128)
    dst_p = jnp.concatenate([edge_index[1].astype(i32),
                             jnp.full((EP - E,), DUMMY_ROW, i32)]).reshape(32, 5, 128)
    ti0_p = jnp.concatenate([target_index[0].astype(i32),
                             jnp.zeros((TIP - N,), i32)]).reshape(32, 3, 128)
    ti1_p = jnp.concatenate([target_index[1].astype(i32),
                             jnp.zeros((TIP - N,), i32)]).reshape(32, 3, 128)
    bat_p = jnp.concatenate([batch.astype(i32),
                             jnp.zeros((TIP - N,), i32)]).reshape(32, 3, 128)
    ones_ep = jnp.ones((EP, 128), f32)
    zeros328 = jnp.zeros((328, 128), f32)
    batch2d = batch.astype(i32).reshape(N, 1)
    tc2d = target_class.astype(i32).reshape(N, 1)

    out = _lin_bn(x, p["pre_W1"], p["pre_g1"], p["pre_b1"], True)
    out = _lin_bn(out, p["pre_W2"], p["pre_g2"], p["pre_b2"], True)
    h = out

    ea = jnp.pad(edge_attr, ((0, 0), (0, D - ED)))
    w1 = jnp.pad(p["enc_W1"], ((0, D - ED), (0, 0)))
    e = _lin_bn(ea, w1, p["enc_g1"], p["enc_b1"], True)
    e = _lin_bn(e, p["enc_W2"], p["enc_g2"], p["enc_b2"], True)
    e3 = _lin_bn(e, p["enc_W3"], p["enc_g3"], p["enc_b3"], True)

    ebar, gram = _full_call(
        _stats_body,
        (jax.ShapeDtypeStruct((1, D), f32), jax.ShapeDtypeStruct((D, D), f32)),
        e3)
    tp2, s4v, cmat = pl.pallas_call(
        _enc4_body,
        grid=(4,),
        in_specs=[
            pl.BlockSpec((1, D), lambda i: (0, 0)),
            pl.BlockSpec((D, D), lambda i: (0, 0)),
            pl.BlockSpec((D, 4096), lambda i: (0, i)),
            pl.BlockSpec((1, 4096), lambda i: (0, i)),
            pl.BlockSpec((1, 4096), lambda i: (0, i)),
        ],
        out_specs=(pl.BlockSpec((D, 4096), lambda i: (0, i)),
                   pl.BlockSpec((1, 4096), lambda i: (0, i)),
                   pl.BlockSpec((32, D), lambda i: (i, 0))),
        out_shape=(jax.ShapeDtypeStruct((D, D * D), f32),
                   jax.ShapeDtypeStruct((1, D * D), f32),
                   jax.ShapeDtypeStruct((D, D), f32)),
        compiler_params=_CP,
    )(ebar, gram, p["enc_W4"], p["enc_g4"].reshape(1, -1),
      p["enc_b4"].reshape(1, -1))

    e3p = jnp.pad(e3, ((0, EP - E), (0, 0)))

    cntf = _sc_scatter_add(ones_ep, dst_p, zeros328).reshape(NP, 128)

    wih2, wihn = p["gru_Wih"][:, :2 * D], p["gru_Wih"][:, 2 * D:]
    whh2, whhn = p["gru_Whh"][:, :2 * D], p["gru_Whh"][:, 2 * D:]
    b2 = (p["gru_bih"][:2 * D] + p["gru_bhh"][:2 * D]).reshape(1, -1)
    bihn = p["gru_bih"][2 * D:].reshape(1, -1)
    bhhn = p["gru_bhh"][2 * D:].reshape(1, -1)

    for _ in range(3):
        ost = _sc_gather(out, src_p, 5, 128)
        msg = pl.pallas_call(
            _msg_body,
            grid=(EP // 128,),
            in_specs=[
                pl.BlockSpec((128, D), lambda i: (i, 0)),
                pl.BlockSpec((128, D), lambda i: (i, 0)),
                pl.BlockSpec((D, D * D), lambda i: (0, 0)),
                pl.BlockSpec((1, D * D), lambda i: (0, 0)),
                pl.BlockSpec((D, D), lambda i: (0, 0)),
            ],
            out_specs=pl.BlockSpec((128, D), lambda i: (i, 0)),
            out_shape=jax.ShapeDtypeStruct((EP, D), f32),
            compiler_params=_CP,
        )(e3p, ost, tp2, s4v, cmat)
        aggf = _sc_scatter_add(msg, dst_p, zeros328).reshape(NP, 128)
        m = pl.pallas_call(
            _premsg_body,
            grid=(5,),
            in_specs=[pl.BlockSpec((2000, D), lambda i: (i, 0)),
                      pl.BlockSpec((2000, D), lambda i: (i, 0)),
                      pl.BlockSpec((1, D), lambda i: (0, 0))],
            out_specs=pl.BlockSpec((2000, D), lambda i: (i, 0)),
            out_shape=jax.ShapeDtypeStruct((N, D), f32),
            compiler_params=_CP,
        )(aggf, cntf, p["conv_b"].reshape(1, -1))
        h = pl.pallas_call(
            _gru_body,
            grid=(5,),
            in_specs=[pl.BlockSpec((2000, D), lambda i: (i, 0)),
                      pl.BlockSpec((2000, D), lambda i: (i, 0)),
                      pl.BlockSpec((D, 2 * D), lambda i: (0, 0)),
                      pl.BlockSpec((D, 2 * D), lambda i: (0, 0)),
                      pl.BlockSpec((1, 2 * D), lambda i: (0, 0)),
                      pl.BlockSpec((D, D), lambda i: (0, 0)),
                      pl.BlockSpec((D, D), lambda i: (0, 0)),
                      pl.BlockSpec((1, D), lambda i: (0, 0)),
                      pl.BlockSpec((1, D), lambda i: (0, 0))],
            out_specs=pl.BlockSpec((2000, D), lambda i: (i, 0)),
            out_shape=jax.ShapeDtypeStruct((N, D), f32),
            compiler_params=_CP,
        )(m, h, wih2, whh2, b2, wihn, whhn, bihn, bhhn)
        out = h

    q_star = pl.pallas_call(
        _set2set_body,
        out_shape=jax.ShapeDtypeStruct((B, 2 * D), f32),
        scratch_shapes=[pltpu.VMEM((N, 1), f32), pltpu.VMEM((N, 1), f32)],
        compiler_params=_CP,
    )(out, batch2d, p["lstm_Wih"], p["lstm_Whh"],
      p["lstm_bih"].reshape(1, -1), p["lstm_bhh"].reshape(1, -1))

    n0 = _sc_gather(out, ti0_p, 3, 128)[:N]
    n1 = _sc_gather(out, ti1_p, 3, 128)[:N]
    s2s = _sc_gather(q_star, bat_p, 3, 256)[:N]

    w1a = p["pred_W1"][:D]
    w1b = p["pred_W1"][D:2 * D]
    w1c = p["pred_W1"][2 * D:]
    a1 = _lin_bn_multi([n0, n1, s2s], [w1a, w1b, w1c],
                       p["pred_g1"], p["pred_b1"], True)
    a2 = _lin_bn(a1, p["pred_W2"], p["pred_g2"], p["pred_b2"], True, rc=1000)

    w3p = jnp.pad(p["pred_W3"], ((0, 0), (0, D - NT)))
    b3p = jnp.pad(p["pred_b3"], ((0, D - NT),)).reshape(1, -1)
    pred = pl.pallas_call(
        _pred3_body,
        grid=(5,),
        in_specs=[pl.BlockSpec((2000, 4 * D), lambda i: (i, 0)),
                  pl.BlockSpec((4 * D, D), lambda i: (0, 0)),
                  pl.BlockSpec((1, D), lambda i: (0, 0)),
                  pl.BlockSpec((2000, 1), lambda i: (i, 0))],
        out_specs=pl.BlockSpec((2000, 1), lambda i: (i, 0)),
        out_shape=jax.ShapeDtypeStruct((N, 1), f32),
        compiler_params=_CP,
    )(a2, w3p, b3p, tc2d)
    return pred.reshape(N)


# speedup vs baseline: 1.0116x; 1.0116x over previous
import functools

import jax
import jax.numpy as jnp
from jax import lax
from jax.experimental import pallas as pl
from jax.experimental.pallas import tpu as pltpu
from jax.experimental.pallas import tpu_sc as plsc

N, E, D, ED, B, NT = 10000, 20000, 128, 4, 512, 8
EP = 20480
NP = 10240
TIP = 12288
DUMMY_ROW = 10200

_CP = pltpu.CompilerParams(vmem_limit_bytes=100 * 1024 * 1024)


_HI = lax.Precision.HIGHEST


def _r(x):
    return x.astype(jnp.bfloat16).astype(jnp.float32)


def _rb(x):
    return x.astype(jnp.bfloat16)


def _lin_bn_body(*refs, act, nrows, n_in, rc):
    a_refs = refs[:n_in]
    w_refs = refs[n_in:2 * n_in]
    g_ref, b_ref, o_ref, y_s = refs[2 * n_in:]
    ws = [_rb(w_ref[...]) for w_ref in w_refs]
    nch = nrows // rc

    def pass1(ci, carry):
        s, ss = carry
        y = jnp.dot(_rb(a_refs[0][pl.ds(ci * rc, rc), :]), ws[0],
                    preferred_element_type=jnp.float32)
        for j in range(1, n_in):
            y = y + jnp.dot(_rb(a_refs[j][pl.ds(ci * rc, rc), :]), ws[j],
                            preferred_element_type=jnp.float32)
        y_s[pl.ds(ci * rc, rc), :] = y
        return (s + jnp.sum(y, axis=0, keepdims=True),
                ss + jnp.sum(y * y, axis=0, keepdims=True))

    ct = ws[0].shape[1]
    s, ss = lax.fori_loop(0, nch, pass1, (jnp.zeros((1, ct), jnp.float32),
                                          jnp.zeros((1, ct), jnp.float32)))
    m = s / float(nrows)
    v = ss / float(nrows) - m * m
    sc = g_ref[...] / jnp.sqrt(v + 1e-5)
    off = b_ref[...] - m * sc

    def pass2(ci, _):
        y = y_s[pl.ds(ci * rc, rc), :] * sc + off
        if act:
            y = jnp.maximum(y, 0.0)
        o_ref[pl.ds(ci * rc, rc), :] = y
        return 0

    lax.fori_loop(0, nch, pass2, 0)


def _lin_bn_multi(a_list, w_list, g, b, act, col_tile=128, rc=2000):
    rows = a_list[0].shape[0]
    cols = w_list[0].shape[1]
    nt = cols // col_tile
    n_in = len(a_list)
    in_specs = ([pl.BlockSpec(a.shape, lambda i: (0, 0)) for a in a_list]
                + [pl.BlockSpec((w.shape[0], col_tile), lambda i: (0, i))
                   for w in w_list]
                + [pl.BlockSpec((1, col_tile), lambda i: (0, i)),
                   pl.BlockSpec((1, col_tile), lambda i: (0, i))])
    return pl.pallas_call(
        functools.partial(_lin_bn_body, act=act, nrows=rows, n_in=n_in, rc=rc),
        grid=(nt,),
        in_specs=in_specs,
        out_specs=pl.BlockSpec((rows, col_tile), lambda i: (0, i)),
        out_shape=jax.ShapeDtypeStruct((rows, cols), jnp.float32),
        scratch_shapes=[pltpu.VMEM((rows, col_tile), jnp.float32)],
        compiler_params=_CP,
    )(*a_list, *w_list, g.reshape(1, -1), b.reshape(1, -1))


def _lin_bn(a, w, g, b, act, col_tile=128, rc=2000):
    return _lin_bn_multi([a], [w], g, b, act, col_tile, rc)


def _stats_body(e3_ref, ebar_ref, gram_ref):
    rc = 2000
    nch = E // rc

    def acc(ci, carry):
        s, gm = carry
        e3 = _r(e3_ref[pl.ds(ci * rc, rc), :])
        return (s + jnp.sum(e3, axis=0, keepdims=True),
                gm + lax.dot_general(e3, e3, (((0,), (0,)), ((), ())),
                                     preferred_element_type=jnp.float32,
                                     precision=_HI))

    s, gm = lax.fori_loop(0, nch, acc, (jnp.zeros((1, D), jnp.float32),
                                        jnp.zeros((D, D), jnp.float32)))
    ebar_ref[...] = s / float(E)
    gram_ref[...] = gm / float(E)


def _enc4_body(ebar_ref, gram_ref, w4_ref, g4_ref, b4_ref,
               tp2_ref, s4_ref, cmat_ref):
    ebar = ebar_ref[...]
    cov = gram_ref[...] - lax.dot_general(
        ebar, ebar, (((0,), (0,)), ((), ())),
        preferred_element_type=jnp.float32, precision=_HI)
    w4 = _r(w4_ref[...])
    m4 = jnp.dot(ebar, w4, preferred_element_type=jnp.float32, precision=_HI)
    q = jnp.dot(cov, w4, preferred_element_type=jnp.float32, precision=_HI)
    v4 = jnp.sum(q * w4, axis=0, keepdims=True)
    s4 = g4_ref[...] / jnp.sqrt(v4 + 1e-5)
    tp2_ref[...] = w4.astype(jnp.bfloat16)
    s4_ref[...] = s4
    cmat_ref[...] = (b4_ref[...] - m4 * s4).reshape(32, D)


def _msg_body(e3_ref, ost_ref, tp2_ref, s4_ref, cmat_ref, msg_ref):
    te = ost_ref.shape[0]
    ost = _r(ost_ref[...])
    p = jnp.dot(e3_ref[...], tp2_ref[...],
                preferred_element_type=jnp.float32)
    s43 = s4_ref[...].reshape(1, D, D)
    cm3 = cmat_ref[...].reshape(1, D, D)
    w_e = _r(p.reshape(te, D, D) * s43 + cm3)
    msg_ref[...] = jnp.sum(w_e * ost[:, :, None], axis=1)


def _premsg_body(pp_ref, cp_ref, cb_ref, m_ref):
    cnt = jnp.maximum(cp_ref[...], 1.0)
    m_ref[...] = jnp.maximum(pp_ref[...] / cnt + cb_ref[...], 0.0)


def _gru_body(m_ref, h_ref, wih2_ref, whh2_ref, b2_ref, wihn_ref, whhn_ref,
              bihn_ref, bhhn_ref, o_ref):
    m = _rb(m_ref[...])
    h = h_ref[...]
    hb = _rb(h)
    g2 = (jnp.dot(m, _rb(wih2_ref[...]), preferred_element_type=jnp.float32)
          + jnp.dot(hb, _rb(whh2_ref[...]), preferred_element_type=jnp.float32)
          + b2_ref[...])
    r = jax.nn.sigmoid(g2[:, :D])
    z = jax.nn.sigmoid(g2[:, D:])
    gxn = jnp.dot(m, _rb(wihn_ref[...]), preferred_element_type=jnp.float32) + bihn_ref[...]
    ghn = jnp.dot(hb, _rb(whhn_ref[...]), preferred_element_type=jnp.float32) + bhhn_ref[...]
    n = jnp.tanh(gxn + r * ghn)
    o_ref[...] = (1.0 - z) * n + z * h


_S2S_NC = 2000


def _set2set_body(out_ref, batch_ref, wih_ref, whh_ref, bih_ref, bhh_ref,
                  qs_ref, ee_s, ex_s):
    f32 = jnp.float32
    nc = _S2S_NC
    nch = N // nc
    iota_b = lax.broadcasted_iota(jnp.int32, (nc, B), 1)

    def _mask(ci):
        bat = batch_ref[pl.ds(ci * nc, nc), :]
        return (bat == iota_b).astype(f32)

    hl = jnp.zeros((B, D), f32)
    cl = jnp.zeros((B, D), f32)
    q_star = jnp.zeros((B, 2 * D), f32)
    for _ in range(6):
        g_all = (jnp.dot(_rb(q_star), _rb(wih_ref[...]), preferred_element_type=f32)
                 + bih_ref[...]
                 + jnp.dot(_rb(hl), _rb(whh_ref[...]), preferred_element_type=f32)
                 + bhh_ref[...])
        ig = jax.nn.sigmoid(g_all[:, :D])
        fg = jax.nn.sigmoid(g_all[:, D:2 * D])
        gg = jnp.tanh(g_all[:, 2 * D:3 * D])
        og = jax.nn.sigmoid(g_all[:, 3 * D:])
        cl = fg * cl + ig * gg
        hl = og * jnp.tanh(cl)
        q = hl

        def p1(ci, emax):
            m = _mask(ci)
            outc = out_ref[pl.ds(ci * nc, nc), :]
            qb = jnp.dot(m, q, preferred_element_type=f32, precision=lax.Precision.HIGHEST)
            ee = jnp.sum(outc * qb, axis=1, keepdims=True)
            ee_s[pl.ds(ci * nc, nc), :] = ee
            vals = m * ee + (m - 1.0) * 1e30
            return jnp.maximum(emax, jnp.max(vals, axis=0, keepdims=True))

        emax = lax.fori_loop(0, nch, p1, jnp.full((1, B), -1e30, f32))

        def p2(ci, esum):
            m = _mask(ci)
            emaxb = jnp.sum(m * emax, axis=1, keepdims=True)
            ex = jnp.exp(ee_s[pl.ds(ci * nc, nc), :] - emaxb)
            ex_s[pl.ds(ci * nc, nc), :] = ex
            return esum + jnp.sum(m * ex, axis=0, keepdims=True)

        esum = lax.fori_loop(0, nch, p2, jnp.zeros((1, B), f32))

        def p3(ci, rr):
            m = _mask(ci)
            esb = jnp.sum(m * esum, axis=1, keepdims=True)
            a = ex_s[pl.ds(ci * nc, nc), :] / (esb + 1e-16)
            outc = out_ref[pl.ds(ci * nc, nc), :]
            return rr + lax.dot_general(m, a * outc, (((0,), (0,)), ((), ())),
                                        preferred_element_type=f32, precision=lax.Precision.HIGHEST)

        rr = lax.fori_loop(0, nch, p3, jnp.zeros((B, D), f32))
        q_star = jnp.concatenate([q, rr], axis=1)
    qs_ref[...] = q_star


def _pred3_body(a_ref, w_ref, b_ref, tc_ref, o_ref):
    rows = a_ref.shape[0]
    y = jnp.dot(_rb(a_ref[...]), _rb(w_ref[...]),
                preferred_element_type=jnp.float32) + b_ref[...]
    sel = (tc_ref[...] == lax.broadcasted_iota(jnp.int32, (rows, D), 1))
    o_ref[...] = jnp.sum(jnp.where(sel, y, 0.0), axis=1, keepdims=True)


_MESH = plsc.VectorSubcoreMesh(core_axis_name="c", subcore_axis_name="s")


def _sc_gather(table, idx3, nj, dm):
    rows_out = 32 * nj * 128

    @functools.partial(
        pl.kernel, mesh=_MESH,
        out_type=jax.ShapeDtypeStruct((rows_out, dm), jnp.float32),
        scratch_types=[
            pltpu.VMEM((nj, 128), jnp.int32),
            pltpu.VMEM((nj * 128, dm), jnp.float32),
            pltpu.SemaphoreType.DMA,
        ],
    )
    def gk(table_hbm, idx_hbm, out_hbm, idx_v, rows_v, sem):
        c = lax.axis_index("c")
        s = lax.axis_index("s")
        wid = s * 2 + c
        pltpu.sync_copy(idx_hbm.at[wid], idx_v)
        cps = [pltpu.async_copy(table_hbm.at[idx_v.at[j]],
                                rows_v.at[pl.ds(j * 128, 128)], sem)
               for j in range(nj)]
        for cp in cps:
            cp.wait()
        pltpu.sync_copy(rows_v, out_hbm.at[pl.ds(wid * nj * 128, nj * 128)])

    return gk(table, idx3)


NPH = 5120
SPAD = 5248


def _sc_scatter_add(vals, dst3, zeros328):
    @functools.partial(
        pl.kernel, mesh=_MESH,
        out_type=jax.ShapeDtypeStruct((2, NPH, 128), jnp.float32),
        scratch_types=[
            pltpu.VMEM((5, 128), jnp.int32),
            pltpu.VMEM((640, 128), jnp.float32),
            pltpu.VMEM_SHARED((SPAD, 128), jnp.float32),
        ],
    )
    def sk(vals_hbm, dst_hbm, z_hbm, out_hbm, idx_v, buf, shared):
        c = lax.axis_index("c")
        s = lax.axis_index("s")
        lo = c * NPH
        pltpu.sync_copy(z_hbm, shared.at[pl.ds(s * 328, 328)])
        plsc.subcore_barrier()
        for half in range(2):
            r = half * 16 + s
            pltpu.sync_copy(dst_hbm.at[r], idx_v)
            for j in range(5):
                for l in range(8):
                    v = idx_v[j, pl.ds(l * 16, 16)] - lo
                    ok = (v >= 0) & (v < NPH)
                    idx_v[j, pl.ds(l * 16, 16)] = jnp.where(ok, v, NPH)
            pltpu.sync_copy(vals_hbm.at[pl.ds(r * 640, 640)], buf)
            for j in range(5):
                pltpu.sync_copy(buf.at[pl.ds(j * 128, 128)],
                                shared.at[idx_v.at[j]], add=True)
        plsc.subcore_barrier()
        pltpu.sync_copy(shared.at[pl.ds(s * 320, 320)], buf.at[pl.ds(0, 320)])
        pltpu.sync_copy(buf.at[pl.ds(0, 320)], out_hbm.at[c, pl.ds(s * 320, 320)])

    return sk(vals, dst3, zeros328)


def _full_call(body, out_shapes, *args):
    return pl.pallas_call(body, out_shape=out_shapes, compiler_params=_CP)(*args)


def kernel(x, edge_attr, params, edge_index, target_index, batch, target_class):
    p = params
    f32, i32 = jnp.float32, jnp.int32

    src_p = jnp.concatenate([edge_index[0].astype(i32),
                             jnp.zeros((EP - E,), i32)]).reshape(32, 5, 128)
    dst_p = jnp.concatenate([edge_index[1].astype(i32),
                             jnp.full((EP - E,), DUMMY_ROW, i32)]).reshape(32, 5, 128)
    ti0_p = jnp.concatenate([target_index[0].astype(i32),
                             jnp.zeros((TIP - N,), i32)]).reshape(32, 3, 128)
    ti1_p = jnp.concatenate([target_index[1].astype(i32),
                             jnp.zeros((TIP - N,), i32)]).reshape(32, 3, 128)
    bat_p = jnp.concatenate([batch.astype(i32),
                             jnp.zeros((TIP - N,), i32)]).reshape(32, 3, 128)
    ones_ep = jnp.ones((EP, 128), f32)
    zeros328 = jnp.zeros((328, 128), f32)
    batch2d = batch.astype(i32).reshape(N, 1)
    tc2d = target_class.astype(i32).reshape(N, 1)

    out = _lin_bn(x, p["pre_W1"], p["pre_g1"], p["pre_b1"], True)
    out = _lin_bn(out, p["pre_W2"], p["pre_g2"], p["pre_b2"], True)
    h = out

    ea = jnp.pad(edge_attr, ((0, 0), (0, D - ED)))
    w1 = jnp.pad(p["enc_W1"], ((0, D - ED), (0, 0)))
    e = _lin_bn(ea, w1, p["enc_g1"], p["enc_b1"], True)
    e = _lin_bn(e, p["enc_W2"], p["enc_g2"], p["enc_b2"], True)
    e3 = _lin_bn(e, p["enc_W3"], p["enc_g3"], p["enc_b3"], True)

    ebar, gram = _full_call(
        _stats_body,
        (jax.ShapeDtypeStruct((1, D), f32), jax.ShapeDtypeStruct((D, D), f32)),
        e3)
    tp2, s4v, cmat = pl.pallas_call(
        _enc4_body,
        grid=(4,),
        in_specs=[
            pl.BlockSpec((1, D), lambda i: (0, 0)),
            pl.BlockSpec((D, D), lambda i: (0, 0)),
            pl.BlockSpec((D, 4096), lambda i: (0, i)),
            pl.BlockSpec((1, 4096), lambda i: (0, i)),
            pl.BlockSpec((1, 4096), lambda i: (0, i)),
        ],
        out_specs=(pl.BlockSpec((D, 4096), lambda i: (0, i)),
                   pl.BlockSpec((1, 4096), lambda i: (0, i)),
                   pl.BlockSpec((32, D), lambda i: (i, 0))),
        out_shape=(jax.ShapeDtypeStruct((D, D * D), jnp.bfloat16),
                   jax.ShapeDtypeStruct((1, D * D), f32),
                   jax.ShapeDtypeStruct((D, D), f32)),
        compiler_params=_CP,
    )(ebar, gram, p["enc_W4"], p["enc_g4"].reshape(1, -1),
      p["enc_b4"].reshape(1, -1))

    e3p = jnp.pad(e3, ((0, EP - E), (0, 0))).astype(jnp.bfloat16)

    cntf = _sc_scatter_add(ones_ep, dst_p, zeros328).reshape(NP, 128)

    wih2, wihn = p["gru_Wih"][:, :2 * D], p["gru_Wih"][:, 2 * D:]
    whh2, whhn = p["gru_Whh"][:, :2 * D], p["gru_Whh"][:, 2 * D:]
    b2 = (p["gru_bih"][:2 * D] + p["gru_bhh"][:2 * D]).reshape(1, -1)
    bihn = p["gru_bih"][2 * D:].reshape(1, -1)
    bhhn = p["gru_bhh"][2 * D:].reshape(1, -1)

    for _ in range(3):
        ost = _sc_gather(out, src_p, 5, 128)
        msg = pl.pallas_call(
            _msg_body,
            grid=(EP // 128,),
            in_specs=[
                pl.BlockSpec((128, D), lambda i: (i, 0)),
                pl.BlockSpec((128, D), lambda i: (i, 0)),
                pl.BlockSpec((D, D * D), lambda i: (0, 0)),
                pl.BlockSpec((1, D * D), lambda i: (0, 0)),
                pl.BlockSpec((D, D), lambda i: (0, 0)),
            ],
            out_specs=pl.BlockSpec((128, D), lambda i: (i, 0)),
            out_shape=jax.ShapeDtypeStruct((EP, D), f32),
            compiler_params=_CP,
        )(e3p, ost, tp2, s4v, cmat)
        aggf = _sc_scatter_add(msg, dst_p, zeros328).reshape(NP, 128)
        m = pl.pallas_call(
            _premsg_body,
            grid=(5,),
            in_specs=[pl.BlockSpec((2000, D), lambda i: (i, 0)),
                      pl.BlockSpec((2000, D), lambda i: (i, 0)),
                      pl.BlockSpec((1, D), lambda i: (0, 0))],
            out_specs=pl.BlockSpec((2000, D), lambda i: (i, 0)),
            out_shape=jax.ShapeDtypeStruct((N, D), f32),
            compiler_params=_CP,
        )(aggf, cntf, p["conv_b"].reshape(1, -1))
        h = pl.pallas_call(
            _gru_body,
            grid=(5,),
            in_specs=[pl.BlockSpec((2000, D), lambda i: (i, 0)),
                      pl.BlockSpec((2000, D), lambda i: (i, 0)),
                      pl.BlockSpec((D, 2 * D), lambda i: (0, 0)),
                      pl.BlockSpec((D, 2 * D), lambda i: (0, 0)),
                      pl.BlockSpec((1, 2 * D), lambda i: (0, 0)),
                      pl.BlockSpec((D, D), lambda i: (0, 0)),
                      pl.BlockSpec((D, D), lambda i: (0, 0)),
                      pl.BlockSpec((1, D), lambda i: (0, 0)),
                      pl.BlockSpec((1, D), lambda i: (0, 0))],
            out_specs=pl.BlockSpec((2000, D), lambda i: (i, 0)),
            out_shape=jax.ShapeDtypeStruct((N, D), f32),
            compiler_params=_CP,
        )(m, h, wih2, whh2, b2, wihn, whhn, bihn, bhhn)
        out = h

    q_star = pl.pallas_call(
        _set2set_body,
        out_shape=jax.ShapeDtypeStruct((B, 2 * D), f32),
        scratch_shapes=[pltpu.VMEM((N, 1), f32), pltpu.VMEM((N, 1), f32)],
        compiler_params=_CP,
    )(out, batch2d, p["lstm_Wih"], p["lstm_Whh"],
      p["lstm_bih"].reshape(1, -1), p["lstm_bhh"].reshape(1, -1))

    n0 = _sc_gather(out, ti0_p, 3, 128)[:N]
    n1 = _sc_gather(out, ti1_p, 3, 128)[:N]
    s2s = _sc_gather(q_star, bat_p, 3, 256)[:N]

    w1a = p["pred_W1"][:D]
    w1b = p["pred_W1"][D:2 * D]
    w1c = p["pred_W1"][2 * D:]
    a1 = _lin_bn_multi([n0, n1, s2s], [w1a, w1b, w1c],
                       p["pred_g1"], p["pred_b1"], True)
    a2 = _lin_bn(a1, p["pred_W2"], p["pred_g2"], p["pred_b2"], True, rc=1000)

    w3p = jnp.pad(p["pred_W3"], ((0, 0), (0, D - NT)))
    b3p = jnp.pad(p["pred_b3"], ((0, D - NT),)).reshape(1, -1)
    pred = pl.pallas_call(
        _pred3_body,
        grid=(5,),
        in_specs=[pl.BlockSpec((2000, 4 * D), lambda i: (i, 0)),
                  pl.BlockSpec((4 * D, D), lambda i: (0, 0)),
                  pl.BlockSpec((1, D), lambda i: (0, 0)),
                  pl.BlockSpec((2000, 1), lambda i: (i, 0))],
        out_specs=pl.BlockSpec((2000, 1), lambda i: (i, 0)),
        out_shape=jax.ShapeDtypeStruct((N, 1), f32),
        compiler_params=_CP,
    )(a2, w3p, b3p, tc2d)
    return pred.reshape(N)
